# Initial kernel scaffold; baseline (speedup 1.0000x reference)
import jax
import jax.numpy as jnp
from jax.experimental import pallas as pl


def kernel(x, edge_index, W1, asrc1, adst1, b1, W2, asrc2, adst2, b2, W3, asrc3, adst3, b3):
    raise NotImplementedError("write your pallas kernel here")


# speedup vs baseline: 37.5866x; 37.5866x over previous
import functools

import jax
import jax.numpy as jnp
from jax import lax
from jax.experimental import pallas as pl
from jax.experimental.pallas import tpu as pltpu
from jax.experimental.pallas import tpu_sc as plsc

NN = 100000
FP = 1536
MB = 400
CHUNK = 400
NCHUNK = 250
NWORK = 32
CPW = 8
KE = 128
ACC_R = CHUNK + 8
BNDP = 272
WROW = 128
BIGDST = 1 << 30


def _sc_edge_kernel(dcols, heads, chs, asrc_col, adl, ex_col):
    wrow = WROW
    mesh = plsc.VectorSubcoreMesh(core_axis_name="c", subcore_axis_name="s")
    groups = KE // 16

    @functools.partial(
        pl.kernel,
        out_type=jax.ShapeDtypeStruct((NN, wrow), jnp.float32),
        mesh=mesh,
        compiler_params=pltpu.CompilerParams(needs_layout_passes=False),
        scratch_types=[
            pltpu.VMEM((KE, wrow), jnp.float32),
            pltpu.VMEM((KE,), jnp.int32),
            pltpu.VMEM((KE,), jnp.int32),
            pltpu.VMEM((KE,), jnp.int32),
            pltpu.VMEM((ACC_R, wrow), jnp.float32),
            pltpu.VMEM((BNDP,), jnp.int32),
            pltpu.VMEM_SHARED((16 * ACC_R, wrow), jnp.float32),
            pltpu.SemaphoreType.DMA,
        ],
    )
    def edge_kernel(hext, srcs, dsts, bnd, zrs, out, rowbuf, srcbuf, dstbuf,
                    dlbuf, chunkbuf, bndv, acc, sem):
        cid = lax.axis_index("c")
        sid = lax.axis_index("s")
        wid = sid * 2 + cid
        accbase = sid * ACC_R

        pltpu.sync_copy(bnd, bndv)

        for ci in range(CPW):
            cc = ci * NWORK + wid

            @pl.when(cc < NCHUNK)
            def _chunk():
                base = cc * CHUNK
                pltpu.sync_copy(zrs, acc.at[pl.ds(accbase, ACC_R)])
                pltpu.sync_copy(hext.at[pl.ds(base, CHUNK)],
                                chunkbuf.at[pl.ds(0, CHUNK)])
                bpair = bndv[pl.ds(cc, 16)]
                start = bpair[0]
                end = bpair[1]
                eb8 = (start // 8) * 8
                nb = (end - eb8 + KE - 1) // KE

                def block(b, _):
                    eb = eb8 + b * KE
                    pltpu.sync_copy(srcs.at[pl.ds(eb, KE)], srcbuf)
                    pltpu.sync_copy(dsts.at[pl.ds(eb, KE)], dstbuf)
                    pltpu.async_copy(hext.at[srcbuf], rowbuf, sem).wait()

                    def group(g, _):
                        off = pl.multiple_of(g * 16, 16)
                        e16 = lax.iota(jnp.int32, 16) + off
                        dv = dstbuf[pl.ds(off, 16)]
                        dl = dv - base
                        dl = jnp.where(dl < 0, CHUNK, dl)
                        dl = jnp.minimum(dl, CHUNK)
                        dlbuf[pl.ds(off, 16)] = dl + accbase
                        exs = []
                        for hh in range(heads):
                            a_s = plsc.load_gather(
                                rowbuf,
                                [e16, jnp.full((16,), asrc_col + hh, jnp.int32)])
                            a_d = plsc.load_gather(
                                chunkbuf,
                                [dl, jnp.full((16,), adl + hh, jnp.int32)])
                            s = a_s + a_d
                            alpha = (jnp.maximum(s, 0.0)
                                     + 0.2 * jnp.minimum(s, 0.0))
                            exs.append(jnp.exp(alpha))
                        for hh in range(heads):
                            plsc.store_scatter(
                                rowbuf,
                                [e16, jnp.full((16,), ex_col + hh, jnp.int32)],
                                exs[hh])
                        for col in range(dcols):
                            cvec = jnp.full((16,), col, jnp.int32)
                            v = plsc.load_gather(rowbuf, [e16, cvec])
                            plsc.store_scatter(rowbuf, [e16, cvec],
                                               v * exs[col // chs])
                        return 0
                    lax.fori_loop(0, groups, group, 0)
                    pltpu.sync_copy(rowbuf, acc.at[dlbuf], add=True)
                    return 0
                lax.fori_loop(0, nb, block, 0)
                pltpu.sync_copy(acc.at[pl.ds(accbase, CHUNK)],
                                out.at[pl.ds(base, CHUNK)])

    return edge_kernel


_sc_edge_64 = _sc_edge_kernel(64, 4, 16, 64, 68, 64)
_sc_edge_42 = _sc_edge_kernel(42, 6, 7, 42, 48, 48)


def _tc1_body(x_ref, w_ref, p_ref, o_ref):
    h = jnp.dot(x_ref[...], w_ref[...], preferred_element_type=jnp.float32)
    o_ref[...] = jnp.dot(h, p_ref[...], preferred_element_type=jnp.float32)


def _tcmid_body(g_ref, r_ref, b_ref, w_ref, p_ref, o_ref):
    g = g_ref[...]
    denx = jnp.dot(g, r_ref[...], preferred_element_type=jnp.float32)
    a = g[:, 0:64] / denx + b_ref[...]
    a = jnp.where(a > 0, a, jnp.exp(a) - 1.0)
    h = jnp.dot(a, w_ref[...], preferred_element_type=jnp.float32)
    o_ref[...] = jnp.dot(h, p_ref[...], preferred_element_type=jnp.float32)


def _tcfin_body(g_ref, r_ref, m_ref, b_ref, o_ref):
    g = g_ref[...]
    denx = jnp.dot(g, r_ref[...], preferred_element_type=jnp.float32)
    z = jnp.dot(g[:, 0:42] / denx, m_ref[...],
                preferred_element_type=jnp.float32) + b_ref[...]
    z = jnp.where(z > 0, z, jnp.exp(z) - 1.0)
    m = jnp.max(z, axis=1, keepdims=True)
    lse = m + jnp.log(jnp.sum(jnp.exp(z - m), axis=1, keepdims=True))
    o_ref[...] = z - lse


def _full_spec(r, c):
    return pl.BlockSpec((r, c), lambda i: (0, 0))


def _sel_cat(att, d):
    h, c = att.shape
    return (att[:, :, None] * jnp.eye(h, dtype=jnp.float32)[:, None, :]
            ).reshape(h * c, h)[:d]


def kernel(x, edge_index, W1, asrc1, adst1, b1, W2, asrc2, adst2, b2,
           W3, asrc3, adst3, b3):
    n = x.shape[0]
    f32 = jnp.float32

    loop = jnp.arange(n, dtype=jnp.int32)
    src = jnp.concatenate([edge_index[0], loop])
    dst = jnp.concatenate([edge_index[1], loop])
    dsort, ssort = lax.sort((dst, src), num_keys=1)
    e2 = src.shape[0]
    src_p = jnp.concatenate([ssort, jnp.zeros((2 * KE,), jnp.int32)])
    dst_p = jnp.concatenate([dsort, jnp.full((2 * KE,), BIGDST, jnp.int32)])
    marks = jnp.arange(NCHUNK + 1, dtype=jnp.int32) * CHUNK
    bnd = jnp.searchsorted(dsort, marks, side="left").astype(jnp.int32)
    bnd_p = jnp.concatenate(
        [bnd, jnp.full((BNDP - NCHUNK - 1,), e2, jnp.int32)])

    i64 = jnp.eye(64, dtype=f32)
    p1 = jnp.concatenate([i64, _sel_cat(asrc1, 64), _sel_cat(adst1, 64),
                          jnp.zeros((64, WROW - 72), f32)], axis=1)
    p2 = jnp.concatenate([i64, _sel_cat(asrc2, 64), _sel_cat(adst2, 64),
                          jnp.zeros((64, WROW - 72), f32)], axis=1)
    i42 = jnp.eye(42, dtype=f32)
    p3 = jnp.concatenate([i42, _sel_cat(asrc3, 42), _sel_cat(adst3, 42),
                          jnp.zeros((42, WROW - 54), f32)], axis=1)
    rep16 = jnp.kron(jnp.eye(4, dtype=f32), jnp.ones((1, 16), f32))
    rext = jnp.concatenate([jnp.zeros((64, 64), f32), rep16,
                            jnp.zeros((WROW - 68, 64), f32)], axis=0)
    rep7 = jnp.kron(jnp.eye(6, dtype=f32), jnp.ones((1, 7), f32))
    r3 = jnp.concatenate([jnp.zeros((48, 42), f32), rep7,
                          jnp.zeros((WROW - 54, 42), f32)], axis=0)
    m6 = jnp.kron(jnp.ones((6, 1), f32) / 6.0, jnp.eye(7, dtype=f32))
    zrs = jnp.zeros((ACC_R, WROW), f32)

    xp = jnp.pad(x, ((0, 0), (0, FP - x.shape[1])))
    w1p = jnp.pad(W1, ((0, FP - W1.shape[0]), (0, 0)))

    grid = (NN // MB,)
    row_spec = pl.BlockSpec((MB, WROW), lambda i: (i, 0))

    hext1 = pl.pallas_call(
        _tc1_body, grid=grid,
        in_specs=[pl.BlockSpec((MB, FP), lambda i: (i, 0)),
                  _full_spec(FP, 64), _full_spec(64, WROW)],
        out_specs=row_spec,
        out_shape=jax.ShapeDtypeStruct((NN, WROW), f32),
    )(xp, w1p, p1)

    agg1 = _sc_edge_64(hext1, src_p, dst_p, bnd_p, zrs)

    hext2 = pl.pallas_call(
        _tcmid_body, grid=grid,
        in_specs=[row_spec, _full_spec(WROW, 64), _full_spec(1, 64),
                  _full_spec(64, 64), _full_spec(64, WROW)],
        out_specs=row_spec,
        out_shape=jax.ShapeDtypeStruct((NN, WROW), f32),
    )(agg1, rext, b1.reshape(1, 64), W2, p2)

    agg2 = _sc_edge_64(hext2, src_p, dst_p, bnd_p, zrs)

    hext3 = pl.pallas_call(
        _tcmid_body, grid=grid,
        in_specs=[row_spec, _full_spec(WROW, 64), _full_spec(1, 64),
                  _full_spec(64, 42), _full_spec(42, WROW)],
        out_specs=row_spec,
        out_shape=jax.ShapeDtypeStruct((NN, WROW), f32),
    )(agg2, rext, b2.reshape(1, 64), W3, p3)

    agg3 = _sc_edge_42(hext3, src_p, dst_p, bnd_p, zrs)

    out = pl.pallas_call(
        _tcfin_body, grid=grid,
        in_specs=[row_spec, _full_spec(WROW, 42), _full_spec(42, 7),
                  _full_spec(1, 7)],
        out_specs=pl.BlockSpec((MB, 7), lambda i: (i, 0)),
        out_shape=jax.ShapeDtypeStruct((NN, 7), f32),
    )(agg3, r3, m6, b3.reshape(1, 7))

    return out


# speedup vs baseline: 39.3357x; 1.0465x over previous
import functools

import jax
import jax.numpy as jnp
from jax import lax
from jax.experimental import pallas as pl
from jax.experimental.pallas import tpu as pltpu
from jax.experimental.pallas import tpu_sc as plsc

NN = 100000
FP = 1536
MB = 400
CHUNK = 200
NCHUNK = 500
NWORK = 32
CPW = 16
KE = 128
ACC_R = CHUNK + 8
BNDP = 520
WROW = 128
BIGDST = 1 << 30


def _sc_edge_kernel(dcols, heads, chs, asrc_col, adl, ex_col):
    wrow = WROW
    mesh = plsc.VectorSubcoreMesh(core_axis_name="c", subcore_axis_name="s")
    groups = KE // 16

    @functools.partial(
        pl.kernel,
        out_type=jax.ShapeDtypeStruct((NN, wrow), jnp.float32),
        mesh=mesh,
        compiler_params=pltpu.CompilerParams(needs_layout_passes=False),
        scratch_types=[
            pltpu.VMEM((KE, wrow), jnp.float32),
            pltpu.VMEM((KE, wrow), jnp.float32),
            pltpu.VMEM((KE,), jnp.int32),
            pltpu.VMEM((KE,), jnp.int32),
            pltpu.VMEM((KE,), jnp.int32),
            pltpu.VMEM((KE,), jnp.int32),
            pltpu.VMEM((KE,), jnp.int32),
            pltpu.VMEM((KE,), jnp.int32),
            pltpu.VMEM((ACC_R, wrow), jnp.float32),
            pltpu.VMEM((BNDP,), jnp.int32),
            pltpu.VMEM_SHARED((16 * ACC_R, wrow), jnp.float32),
            pltpu.SemaphoreType.DMA,
            pltpu.SemaphoreType.DMA,
        ],
    )
    def edge_kernel(hext, srcs, dsts, bnd, zrs, out, rowa, rowb, sida, sidb,
                    dida, didb, dla, dlb, chunkbuf, bndv, acc, sema, semb):
        cid = lax.axis_index("c")
        sid = lax.axis_index("s")
        wid = sid * 2 + cid
        accbase = sid * ACC_R

        pltpu.sync_copy(bnd, bndv)

        def compute(rowbuf, dlbuf, didbuf, sidbuf, base, sem):
            pltpu.make_async_copy(hext.at[sidbuf], rowbuf, sem).wait()

            def group(g, _):
                off = pl.multiple_of(g * 16, 16)
                e16 = lax.iota(jnp.int32, 16) + off
                dv = didbuf[pl.ds(off, 16)]
                dl = dv - base
                dl = jnp.where(dl < 0, CHUNK, dl)
                dl = jnp.minimum(dl, CHUNK)
                dlbuf[pl.ds(off, 16)] = dl + accbase
                exs = []
                for hh in range(heads):
                    a_s = plsc.load_gather(
                        rowbuf,
                        [e16, jnp.full((16,), asrc_col + hh, jnp.int32)])
                    a_d = plsc.load_gather(
                        chunkbuf,
                        [dl, jnp.full((16,), adl + hh, jnp.int32)])
                    t = a_s + a_d
                    alpha = jnp.maximum(t, 0.0) + 0.2 * jnp.minimum(t, 0.0)
                    exs.append(jnp.exp(alpha))
                for hh in range(heads):
                    plsc.store_scatter(
                        rowbuf,
                        [e16, jnp.full((16,), ex_col + hh, jnp.int32)],
                        exs[hh])
                for col in range(dcols):
                    cvec = jnp.full((16,), col, jnp.int32)
                    v = plsc.load_gather(rowbuf, [e16, cvec])
                    plsc.store_scatter(rowbuf, [e16, cvec],
                                       v * exs[col // chs])
                return 0
            lax.fori_loop(0, KE // 16, group, 0)
            pltpu.sync_copy(rowbuf, acc.at[dlbuf], add=True)

        def chunk_iter(ci, _):
            cc = ci * NWORK + wid

            @pl.when(cc < NCHUNK)
            def _chunk():
                base = cc * CHUNK
                pltpu.sync_copy(zrs, acc.at[pl.ds(accbase, ACC_R)])
                pltpu.sync_copy(hext.at[pl.ds(base, CHUNK)],
                                chunkbuf.at[pl.ds(0, CHUNK)])
                bpair = bndv[pl.ds(cc, 16)]
                start = bpair[0]
                end = bpair[1]
                eb8 = (start // 8) * 8
                npair = (end - eb8 + 2 * KE - 1) // (2 * KE)
                pltpu.sync_copy(srcs.at[pl.ds(eb8, KE)], sida)
                pltpu.sync_copy(dsts.at[pl.ds(eb8, KE)], dida)
                pltpu.async_copy(hext.at[sida], rowa, sema)

                def pair(p, _):
                    eb = eb8 + p * 2 * KE
                    pltpu.sync_copy(srcs.at[pl.ds(eb + KE, KE)], sidb)
                    pltpu.sync_copy(dsts.at[pl.ds(eb + KE, KE)], didb)
                    pltpu.async_copy(hext.at[sidb], rowb, semb)
                    compute(rowa, dla, dida, sida, base, sema)
                    pltpu.sync_copy(srcs.at[pl.ds(eb + 2 * KE, KE)], sida)
                    pltpu.sync_copy(dsts.at[pl.ds(eb + 2 * KE, KE)], dida)
                    pltpu.async_copy(hext.at[sida], rowa, sema)
                    compute(rowb, dlb, didb, sidb, base, semb)
                    return 0
                lax.fori_loop(0, npair, pair, 0)
                pltpu.make_async_copy(hext.at[sida], rowa, sema).wait()
                pltpu.sync_copy(acc.at[pl.ds(accbase, CHUNK)],
                                out.at[pl.ds(base, CHUNK)])
            return 0
        lax.fori_loop(0, CPW, chunk_iter, 0)

    return edge_kernel


_sc_edge_64 = _sc_edge_kernel(64, 4, 16, 64, 68, 64)
_sc_edge_42 = _sc_edge_kernel(42, 6, 7, 42, 48, 48)


def _tc1_body(x_ref, w_ref, p_ref, o_ref):
    h = jnp.dot(x_ref[...], w_ref[...], preferred_element_type=jnp.float32)
    o_ref[...] = jnp.dot(h, p_ref[...], preferred_element_type=jnp.float32)


def _tcmid_body(g_ref, r_ref, b_ref, w_ref, p_ref, o_ref):
    g = g_ref[...]
    denx = jnp.dot(g, r_ref[...], preferred_element_type=jnp.float32)
    a = g[:, 0:64] / denx + b_ref[...]
    a = jnp.where(a > 0, a, jnp.exp(a) - 1.0)
    h = jnp.dot(a, w_ref[...], preferred_element_type=jnp.float32)
    o_ref[...] = jnp.dot(h, p_ref[...], preferred_element_type=jnp.float32)


def _tcfin_body(g_ref, r_ref, m_ref, b_ref, o_ref):
    g = g_ref[...]
    denx = jnp.dot(g, r_ref[...], preferred_element_type=jnp.float32)
    z = jnp.dot(g[:, 0:42] / denx, m_ref[...],
                preferred_element_type=jnp.float32) + b_ref[...]
    z = jnp.where(z > 0, z, jnp.exp(z) - 1.0)
    m = jnp.max(z, axis=1, keepdims=True)
    lse = m + jnp.log(jnp.sum(jnp.exp(z - m), axis=1, keepdims=True))
    o_ref[...] = z - lse


def _full_spec(r, c):
    return pl.BlockSpec((r, c), lambda i: (0, 0))


def _sel_cat(att, d):
    h, c = att.shape
    return (att[:, :, None] * jnp.eye(h, dtype=jnp.float32)[:, None, :]
            ).reshape(h * c, h)[:d]


def kernel(x, edge_index, W1, asrc1, adst1, b1, W2, asrc2, adst2, b2,
           W3, asrc3, adst3, b3):
    n = x.shape[0]
    f32 = jnp.float32

    loop = jnp.arange(n, dtype=jnp.int32)
    src = jnp.concatenate([edge_index[0], loop])
    dst = jnp.concatenate([edge_index[1], loop])
    dsort, ssort = lax.sort((dst, src), num_keys=1)
    e2 = src.shape[0]
    padi = (jnp.arange(4 * KE, dtype=jnp.int32) * 997) % NN
    src_p = jnp.concatenate([ssort, padi])
    dst_p = jnp.concatenate([dsort, jnp.full((4 * KE,), BIGDST, jnp.int32)])
    marks = jnp.arange(NCHUNK + 1, dtype=jnp.int32) * CHUNK
    bnd = jnp.searchsorted(dsort, marks, side="left").astype(jnp.int32)
    bnd_p = jnp.concatenate(
        [bnd, jnp.full((BNDP - NCHUNK - 1,), e2, jnp.int32)])

    i64 = jnp.eye(64, dtype=f32)
    p1 = jnp.concatenate([i64, _sel_cat(asrc1, 64), _sel_cat(adst1, 64),
                          jnp.zeros((64, WROW - 72), f32)], axis=1)
    p2 = jnp.concatenate([i64, _sel_cat(asrc2, 64), _sel_cat(adst2, 64),
                          jnp.zeros((64, WROW - 72), f32)], axis=1)
    i42 = jnp.eye(42, dtype=f32)
    p3 = jnp.concatenate([i42, _sel_cat(asrc3, 42), _sel_cat(adst3, 42),
                          jnp.zeros((42, WROW - 54), f32)], axis=1)
    rep16 = jnp.kron(jnp.eye(4, dtype=f32), jnp.ones((1, 16), f32))
    rext = jnp.concatenate([jnp.zeros((64, 64), f32), rep16,
                            jnp.zeros((WROW - 68, 64), f32)], axis=0)
    rep7 = jnp.kron(jnp.eye(6, dtype=f32), jnp.ones((1, 7), f32))
    r3 = jnp.concatenate([jnp.zeros((48, 42), f32), rep7,
                          jnp.zeros((WROW - 54, 42), f32)], axis=0)
    m6 = jnp.kron(jnp.ones((6, 1), f32) / 6.0, jnp.eye(7, dtype=f32))
    zrs = jnp.zeros((ACC_R, WROW), f32)

    xp = jnp.pad(x, ((0, 0), (0, FP - x.shape[1])))
    w1p = jnp.pad(W1, ((0, FP - W1.shape[0]), (0, 0)))

    grid = (NN // MB,)
    row_spec = pl.BlockSpec((MB, WROW), lambda i: (i, 0))

    hext1 = pl.pallas_call(
        _tc1_body, grid=grid,
        in_specs=[pl.BlockSpec((MB, FP), lambda i: (i, 0)),
                  _full_spec(FP, 64), _full_spec(64, WROW)],
        out_specs=row_spec,
        out_shape=jax.ShapeDtypeStruct((NN, WROW), f32),
    )(xp, w1p, p1)

    agg1 = _sc_edge_64(hext1, src_p, dst_p, bnd_p, zrs)

    hext2 = pl.pallas_call(
        _tcmid_body, grid=grid,
        in_specs=[row_spec, _full_spec(WROW, 64), _full_spec(1, 64),
                  _full_spec(64, 64), _full_spec(64, WROW)],
        out_specs=row_spec,
        out_shape=jax.ShapeDtypeStruct((NN, WROW), f32),
    )(agg1, rext, b1.reshape(1, 64), W2, p2)

    agg2 = _sc_edge_64(hext2, src_p, dst_p, bnd_p, zrs)

    hext3 = pl.pallas_call(
        _tcmid_body, grid=grid,
        in_specs=[row_spec, _full_spec(WROW, 64), _full_spec(1, 64),
                  _full_spec(64, 42), _full_spec(42, WROW)],
        out_specs=row_spec,
        out_shape=jax.ShapeDtypeStruct((NN, WROW), f32),
    )(agg2, rext, b2.reshape(1, 64), W3, p3)

    agg3 = _sc_edge_42(hext3, src_p, dst_p, bnd_p, zrs)

    out = pl.pallas_call(
        _tcfin_body, grid=grid,
        in_specs=[row_spec, _full_spec(WROW, 42), _full_spec(42, 7),
                  _full_spec(1, 7)],
        out_specs=pl.BlockSpec((MB, 7), lambda i: (i, 0)),
        out_shape=jax.ShapeDtypeStruct((NN, 7), f32),
    )(agg3, r3, m6, b3.reshape(1, 7))

    return out


# speedup vs baseline: 40.1950x; 1.0218x over previous
import functools

import jax
import jax.numpy as jnp
from jax import lax
from jax.experimental import pallas as pl
from jax.experimental.pallas import tpu as pltpu
from jax.experimental.pallas import tpu_sc as plsc

NN = 100000
FIN = 1433
MB = 400
CHUNK = 200
NCHUNK = 500
NWORK = 32
CPW = 16
KE = 128
ACC_R = CHUNK + 8
BNDP = 520
WROW = 128
BIGDST = 1 << 30


def _sc_edge_kernel(dcols, heads, chs, asrc_col, adl, ex_col):
    wrow = WROW
    mesh = plsc.VectorSubcoreMesh(core_axis_name="c", subcore_axis_name="s")
    groups = KE // 16

    @functools.partial(
        pl.kernel,
        out_type=jax.ShapeDtypeStruct((NN, wrow), jnp.float32),
        mesh=mesh,
        compiler_params=pltpu.CompilerParams(needs_layout_passes=False),
        scratch_types=[
            pltpu.VMEM((KE, wrow), jnp.float32),
            pltpu.VMEM((KE, wrow), jnp.float32),
            pltpu.VMEM((KE, wrow), jnp.float32),
            pltpu.VMEM((KE, wrow), jnp.float32),
            pltpu.VMEM((KE,), jnp.int32),
            pltpu.VMEM((KE,), jnp.int32),
            pltpu.VMEM((KE,), jnp.int32),
            pltpu.VMEM((KE,), jnp.int32),
            pltpu.VMEM((KE,), jnp.int32),
            pltpu.VMEM((KE,), jnp.int32),
            pltpu.VMEM((KE,), jnp.int32),
            pltpu.VMEM((KE,), jnp.int32),
            pltpu.VMEM((KE,), jnp.int32),
            pltpu.VMEM((KE,), jnp.int32),
            pltpu.VMEM((KE,), jnp.int32),
            pltpu.VMEM((KE,), jnp.int32),
            pltpu.VMEM((ACC_R, wrow), jnp.float32),
            pltpu.VMEM((BNDP,), jnp.int32),
            pltpu.VMEM_SHARED((16 * ACC_R, wrow), jnp.float32),
            pltpu.SemaphoreType.DMA,
            pltpu.SemaphoreType.DMA,
            pltpu.SemaphoreType.DMA,
            pltpu.SemaphoreType.DMA,
        ],
    )
    def edge_kernel(hext, srcs, dsts, bnd, zrs, out,
                    row0, row1, row2, row3, sid0, sid1, sid2, sid3,
                    did0, did1, did2, did3, dl0, dl1, dl2, dl3,
                    chunkbuf, bndv, acc, sem0, sem1, sem2, sem3):
        row = (row0, row1, row2, row3)
        sid = (sid0, sid1, sid2, sid3)
        did = (did0, did1, did2, did3)
        dl = (dl0, dl1, dl2, dl3)
        sem = (sem0, sem1, sem2, sem3)
        cid = lax.axis_index("c")
        scid = lax.axis_index("s")
        wid = scid * 2 + cid
        accbase = scid * ACC_R

        pltpu.sync_copy(bnd, bndv)

        def compute(rowbuf, dlbuf, didbuf, sidbuf, base, sem):
            pltpu.make_async_copy(hext.at[sidbuf], rowbuf, sem).wait()

            def group(g, _):
                off = pl.multiple_of(g * 16, 16)
                e16 = lax.iota(jnp.int32, 16) + off
                dv = didbuf[pl.ds(off, 16)]
                dl = dv - base
                dl = jnp.where(dl < 0, CHUNK, dl)
                dl = jnp.minimum(dl, CHUNK)
                dlbuf[pl.ds(off, 16)] = dl + accbase
                exs = []
                for hh in range(heads):
                    a_s = plsc.load_gather(
                        rowbuf,
                        [e16, jnp.full((16,), asrc_col + hh, jnp.int32)])
                    a_d = plsc.load_gather(
                        chunkbuf,
                        [dl, jnp.full((16,), adl + hh, jnp.int32)])
                    t = a_s + a_d
                    alpha = jnp.maximum(t, 0.0) + 0.2 * jnp.minimum(t, 0.0)
                    exs.append(jnp.exp(alpha))
                for hh in range(heads):
                    plsc.store_scatter(
                        rowbuf,
                        [e16, jnp.full((16,), ex_col + hh, jnp.int32)],
                        exs[hh])
                for col in range(dcols):
                    cvec = jnp.full((16,), col, jnp.int32)
                    v = plsc.load_gather(rowbuf, [e16, cvec])
                    plsc.store_scatter(rowbuf, [e16, cvec],
                                       v * exs[col // chs])
                return 0
            lax.fori_loop(0, KE // 16, group, 0)
            pltpu.sync_copy(rowbuf, acc.at[dlbuf], add=True)

        def chunk_iter(ci, _):
            cc = ci * NWORK + wid

            @pl.when(cc < NCHUNK)
            def _chunk():
                base = cc * CHUNK
                pltpu.sync_copy(zrs, acc.at[pl.ds(accbase, ACC_R)])
                pltpu.sync_copy(hext.at[pl.ds(base, CHUNK)],
                                chunkbuf.at[pl.ds(0, CHUNK)])
                bpair = bndv[pl.ds(cc, 16)]
                start = bpair[0]
                end = bpair[1]
                eb8 = (start // 8) * 8
                nb = (end - eb8 + KE - 1) // KE
                nq = (nb + 3) // 4

                def stage_issue(b, sl):
                    @pl.when(b < nb)
                    def _():
                        eb = eb8 + b * KE
                        pltpu.sync_copy(srcs.at[pl.ds(eb, KE)], sid[sl])
                        pltpu.sync_copy(dsts.at[pl.ds(eb, KE)], did[sl])
                        pltpu.async_copy(hext.at[sid[sl]], row[sl], sem[sl])

                for j in range(3):
                    stage_issue(j, j)

                def quad(qq, _):
                    b0 = qq * 4
                    for j in range(4):
                        stage_issue(b0 + j + 3, (j + 3) % 4)

                        @pl.when(b0 + j < nb)
                        def _():
                            compute(row[j], dl[j], did[j], sid[j], base,
                                    sem[j])
                    return 0
                lax.fori_loop(0, nq, quad, 0)
                pltpu.sync_copy(acc.at[pl.ds(accbase, CHUNK)],
                                out.at[pl.ds(base, CHUNK)])
            return 0
        lax.fori_loop(0, CPW, chunk_iter, 0)

    return edge_kernel


_sc_edge_64 = _sc_edge_kernel(64, 4, 16, 64, 68, 64)
_sc_edge_42 = _sc_edge_kernel(42, 6, 7, 42, 48, 48)


def _tc1_body(x_ref, w_ref, p_ref, o_ref):
    h = jnp.dot(x_ref[...], w_ref[...], preferred_element_type=jnp.float32)
    o_ref[...] = jnp.dot(h, p_ref[...], preferred_element_type=jnp.float32)


def _tcmid_body(g_ref, r_ref, b_ref, w_ref, p_ref, o_ref):
    g = g_ref[...]
    denx = jnp.dot(g, r_ref[...], preferred_element_type=jnp.float32)
    a = g[:, 0:64] / denx + b_ref[...]
    a = jnp.where(a > 0, a, jnp.exp(a) - 1.0)
    h = jnp.dot(a, w_ref[...], preferred_element_type=jnp.float32)
    o_ref[...] = jnp.dot(h, p_ref[...], preferred_element_type=jnp.float32)


def _tcfin_body(g_ref, r_ref, m_ref, b_ref, o_ref):
    g = g_ref[...]
    denx = jnp.dot(g, r_ref[...], preferred_element_type=jnp.float32)
    z = jnp.dot(g[:, 0:42] / denx, m_ref[...],
                preferred_element_type=jnp.float32) + b_ref[...]
    z = jnp.where(z > 0, z, jnp.exp(z) - 1.0)
    m = jnp.max(z, axis=1, keepdims=True)
    lse = m + jnp.log(jnp.sum(jnp.exp(z - m), axis=1, keepdims=True))
    o_ref[...] = z - lse


def _full_spec(r, c):
    return pl.BlockSpec((r, c), lambda i: (0, 0))


def _sel_cat(att, d):
    h, c = att.shape
    return (att[:, :, None] * jnp.eye(h, dtype=jnp.float32)[:, None, :]
            ).reshape(h * c, h)[:d]


def kernel(x, edge_index, W1, asrc1, adst1, b1, W2, asrc2, adst2, b2,
           W3, asrc3, adst3, b3):
    n = x.shape[0]
    f32 = jnp.float32

    loop = jnp.arange(n, dtype=jnp.int32)
    src = jnp.concatenate([edge_index[0], loop])
    dst = jnp.concatenate([edge_index[1], loop])
    dsort, ssort = lax.sort((dst, src), num_keys=1)
    e2 = src.shape[0]
    padi = (jnp.arange(4 * KE, dtype=jnp.int32) * 997) % NN
    src_p = jnp.concatenate([ssort, padi])
    dst_p = jnp.concatenate([dsort, jnp.full((4 * KE,), BIGDST, jnp.int32)])
    marks = jnp.arange(NCHUNK + 1, dtype=jnp.int32) * CHUNK
    bnd = jnp.searchsorted(dsort, marks, side="left").astype(jnp.int32)
    bnd_p = jnp.concatenate(
        [bnd, jnp.full((BNDP - NCHUNK - 1,), e2, jnp.int32)])

    i64 = jnp.eye(64, dtype=f32)
    p1 = jnp.concatenate([i64, _sel_cat(asrc1, 64), _sel_cat(adst1, 64),
                          jnp.zeros((64, WROW - 72), f32)], axis=1)
    p2 = jnp.concatenate([i64, _sel_cat(asrc2, 64), _sel_cat(adst2, 64),
                          jnp.zeros((64, WROW - 72), f32)], axis=1)
    i42 = jnp.eye(42, dtype=f32)
    p3 = jnp.concatenate([i42, _sel_cat(asrc3, 42), _sel_cat(adst3, 42),
                          jnp.zeros((42, WROW - 54), f32)], axis=1)
    rep16 = jnp.kron(jnp.eye(4, dtype=f32), jnp.ones((1, 16), f32))
    rext = jnp.concatenate([jnp.zeros((64, 64), f32), rep16,
                            jnp.zeros((WROW - 68, 64), f32)], axis=0)
    rep7 = jnp.kron(jnp.eye(6, dtype=f32), jnp.ones((1, 7), f32))
    r3 = jnp.concatenate([jnp.zeros((48, 42), f32), rep7,
                          jnp.zeros((WROW - 54, 42), f32)], axis=0)
    m6 = jnp.kron(jnp.ones((6, 1), f32) / 6.0, jnp.eye(7, dtype=f32))
    zrs = jnp.zeros((ACC_R, WROW), f32)


    grid = (NN // MB,)
    row_spec = pl.BlockSpec((MB, WROW), lambda i: (i, 0))

    hext1 = pl.pallas_call(
        _tc1_body, grid=grid,
        in_specs=[pl.BlockSpec((MB, FIN), lambda i: (i, 0)),
                  _full_spec(FIN, 64), _full_spec(64, WROW)],
        out_specs=row_spec,
        out_shape=jax.ShapeDtypeStruct((NN, WROW), f32),
    )(x, W1, p1)

    agg1 = _sc_edge_64(hext1, src_p, dst_p, bnd_p, zrs)

    hext2 = pl.pallas_call(
        _tcmid_body, grid=grid,
        in_specs=[row_spec, _full_spec(WROW, 64), _full_spec(1, 64),
                  _full_spec(64, 64), _full_spec(64, WROW)],
        out_specs=row_spec,
        out_shape=jax.ShapeDtypeStruct((NN, WROW), f32),
    )(agg1, rext, b1.reshape(1, 64), W2, p2)

    agg2 = _sc_edge_64(hext2, src_p, dst_p, bnd_p, zrs)

    hext3 = pl.pallas_call(
        _tcmid_body, grid=grid,
        in_specs=[row_spec, _full_spec(WROW, 64), _full_spec(1, 64),
                  _full_spec(64, 42), _full_spec(42, WROW)],
        out_specs=row_spec,
        out_shape=jax.ShapeDtypeStruct((NN, WROW), f32),
    )(agg2, rext, b2.reshape(1, 64), W3, p3)

    agg3 = _sc_edge_42(hext3, src_p, dst_p, bnd_p, zrs)

    out = pl.pallas_call(
        _tcfin_body, grid=grid,
        in_specs=[row_spec, _full_spec(WROW, 42), _full_spec(42, 7),
                  _full_spec(1, 7)],
        out_specs=pl.BlockSpec((MB, 7), lambda i: (i, 0)),
        out_shape=jax.ShapeDtypeStruct((NN, 7), f32),
    )(agg3, r3, m6, b3.reshape(1, 7))

    return out


# speedup vs baseline: 40.4051x; 1.0052x over previous
import functools

import jax
import jax.numpy as jnp
from jax import lax
from jax.experimental import pallas as pl
from jax.experimental.pallas import tpu as pltpu
from jax.experimental.pallas import tpu_sc as plsc

NN = 100000
FIN = 1433
MB = 400
CHUNK = 200
NCHUNK = 500
NWORK = 32
CPW = 16
KE = 128
ACC_R = CHUNK + 8
BNDP = 520
WROW = 128
BIGDST = 1 << 30


def _sc_edge_kernel(dcols, heads, chs, asrc_col, adl, ex_col):
    wrow = WROW
    mesh = plsc.VectorSubcoreMesh(core_axis_name="c", subcore_axis_name="s")
    groups = KE // 16

    @functools.partial(
        pl.kernel,
        out_type=jax.ShapeDtypeStruct((NN, wrow), jnp.float32),
        mesh=mesh,
        compiler_params=pltpu.CompilerParams(needs_layout_passes=False),
        scratch_types=[
            pltpu.VMEM((KE, wrow), jnp.float32),
            pltpu.VMEM((KE, wrow), jnp.float32),
            pltpu.VMEM((KE, wrow), jnp.float32),
            pltpu.VMEM((KE, wrow), jnp.float32),
            pltpu.VMEM((KE,), jnp.int32),
            pltpu.VMEM((KE,), jnp.int32),
            pltpu.VMEM((KE,), jnp.int32),
            pltpu.VMEM((KE,), jnp.int32),
            pltpu.VMEM((KE,), jnp.int32),
            pltpu.VMEM((KE,), jnp.int32),
            pltpu.VMEM((KE,), jnp.int32),
            pltpu.VMEM((KE,), jnp.int32),
            pltpu.VMEM((KE,), jnp.int32),
            pltpu.VMEM((KE,), jnp.int32),
            pltpu.VMEM((KE,), jnp.int32),
            pltpu.VMEM((KE,), jnp.int32),
            pltpu.VMEM((ACC_R, wrow), jnp.float32),
            pltpu.VMEM((BNDP,), jnp.int32),
            pltpu.VMEM_SHARED((16 * ACC_R, wrow), jnp.float32),
            pltpu.SemaphoreType.DMA,
            pltpu.SemaphoreType.DMA,
            pltpu.SemaphoreType.DMA,
            pltpu.SemaphoreType.DMA,
            pltpu.SemaphoreType.DMA,
            pltpu.SemaphoreType.DMA,
            pltpu.SemaphoreType.DMA,
            pltpu.SemaphoreType.DMA,
        ],
    )
    def edge_kernel(hext, srcs, dsts, bnd, zrs, out,
                    row0, row1, row2, row3, sid0, sid1, sid2, sid3,
                    did0, did1, did2, did3, dl0, dl1, dl2, dl3,
                    chunkbuf, bndv, acc, sem0, sem1, sem2, sem3,
                    ssem0, ssem1, ssem2, ssem3):
        row = (row0, row1, row2, row3)
        sid = (sid0, sid1, sid2, sid3)
        did = (did0, did1, did2, did3)
        dl = (dl0, dl1, dl2, dl3)
        sem = (sem0, sem1, sem2, sem3)
        ssem = (ssem0, ssem1, ssem2, ssem3)
        cid = lax.axis_index("c")
        scid = lax.axis_index("s")
        wid = scid * 2 + cid
        accbase = scid * ACC_R

        pltpu.sync_copy(bnd, bndv)

        def compute(rowbuf, dlbuf, didbuf, sidbuf, base, sem, scat_sem):
            pltpu.make_async_copy(hext.at[sidbuf], rowbuf, sem).wait()

            def group(g, _):
                off = pl.multiple_of(g * 16, 16)
                e16 = lax.iota(jnp.int32, 16) + off
                dv = didbuf[pl.ds(off, 16)]
                dl = dv - base
                dl = jnp.where(dl < 0, CHUNK, dl)
                dl = jnp.minimum(dl, CHUNK)
                dlbuf[pl.ds(off, 16)] = dl + accbase
                exs = []
                for hh in range(heads):
                    a_s = plsc.load_gather(
                        rowbuf,
                        [e16, jnp.full((16,), asrc_col + hh, jnp.int32)])
                    a_d = plsc.load_gather(
                        chunkbuf,
                        [dl, jnp.full((16,), adl + hh, jnp.int32)])
                    t = a_s + a_d
                    alpha = jnp.maximum(t, 0.0) + 0.2 * jnp.minimum(t, 0.0)
                    exs.append(jnp.exp(alpha))
                for hh in range(heads):
                    plsc.store_scatter(
                        rowbuf,
                        [e16, jnp.full((16,), ex_col + hh, jnp.int32)],
                        exs[hh])
                for col in range(dcols):
                    cvec = jnp.full((16,), col, jnp.int32)
                    v = plsc.load_gather(rowbuf, [e16, cvec])
                    plsc.store_scatter(rowbuf, [e16, cvec],
                                       v * exs[col // chs])
                return 0
            lax.fori_loop(0, KE // 16, group, 0)
            pltpu.async_copy(rowbuf, acc.at[dlbuf], scat_sem, add=True)

        def chunk_iter(ci, _):
            cc = ci * NWORK + wid

            @pl.when(cc < NCHUNK)
            def _chunk():
                base = cc * CHUNK
                pltpu.sync_copy(zrs, acc.at[pl.ds(accbase, ACC_R)])
                pltpu.sync_copy(hext.at[pl.ds(base, CHUNK)],
                                chunkbuf.at[pl.ds(0, CHUNK)])
                bpair = bndv[pl.ds(cc, 16)]
                start = bpair[0]
                end = bpair[1]
                eb8 = (start // 8) * 8
                nb = (end - eb8 + KE - 1) // KE
                nq = (nb + 3) // 4

                def stage_issue(b, sl):
                    @pl.when(b < nb)
                    def _():
                        @pl.when(b >= 4)
                        def _():
                            pltpu.make_async_copy(
                                row[sl], acc.at[dl[sl]], ssem[sl]).wait()
                        eb = eb8 + b * KE
                        pltpu.sync_copy(srcs.at[pl.ds(eb, KE)], sid[sl])
                        pltpu.sync_copy(dsts.at[pl.ds(eb, KE)], did[sl])
                        pltpu.async_copy(hext.at[sid[sl]], row[sl], sem[sl])

                for j in range(3):
                    stage_issue(j, j)

                def quad(qq, _):
                    b0 = qq * 4
                    for j in range(4):
                        stage_issue(b0 + j + 3, (j + 3) % 4)

                        @pl.when(b0 + j < nb)
                        def _():
                            compute(row[j], dl[j], did[j], sid[j], base,
                                    sem[j], ssem[j])
                    return 0
                lax.fori_loop(0, nq, quad, 0)
                for sl in range(4):
                    @pl.when(nb > sl)
                    def _():
                        pltpu.make_async_copy(
                            row[sl], acc.at[dl[sl]], ssem[sl]).wait()
                pltpu.sync_copy(acc.at[pl.ds(accbase, CHUNK)],
                                out.at[pl.ds(base, CHUNK)])
            return 0
        lax.fori_loop(0, CPW, chunk_iter, 0)

    return edge_kernel


_sc_edge_64 = _sc_edge_kernel(64, 4, 16, 64, 68, 64)
_sc_edge_42 = _sc_edge_kernel(42, 6, 7, 42, 48, 48)


def _tc1_body(x_ref, w_ref, p_ref, o_ref):
    h = jnp.dot(x_ref[...], w_ref[...], preferred_element_type=jnp.float32)
    o_ref[...] = jnp.dot(h, p_ref[...], preferred_element_type=jnp.float32)


def _tcmid_body(g_ref, r_ref, b_ref, w_ref, p_ref, o_ref):
    g = g_ref[...]
    denx = jnp.dot(g, r_ref[...], preferred_element_type=jnp.float32)
    a = g[:, 0:64] / denx + b_ref[...]
    a = jnp.where(a > 0, a, jnp.exp(a) - 1.0)
    h = jnp.dot(a, w_ref[...], preferred_element_type=jnp.float32)
    o_ref[...] = jnp.dot(h, p_ref[...], preferred_element_type=jnp.float32)


def _tcfin_body(g_ref, r_ref, m_ref, b_ref, o_ref):
    g = g_ref[...]
    denx = jnp.dot(g, r_ref[...], preferred_element_type=jnp.float32)
    z = jnp.dot(g[:, 0:42] / denx, m_ref[...],
                preferred_element_type=jnp.float32) + b_ref[...]
    z = jnp.where(z > 0, z, jnp.exp(z) - 1.0)
    m = jnp.max(z, axis=1, keepdims=True)
    lse = m + jnp.log(jnp.sum(jnp.exp(z - m), axis=1, keepdims=True))
    o_ref[...] = z - lse


def _full_spec(r, c):
    return pl.BlockSpec((r, c), lambda i: (0, 0))


def _sel_cat(att, d):
    h, c = att.shape
    return (att[:, :, None] * jnp.eye(h, dtype=jnp.float32)[:, None, :]
            ).reshape(h * c, h)[:d]


def kernel(x, edge_index, W1, asrc1, adst1, b1, W2, asrc2, adst2, b2,
           W3, asrc3, adst3, b3):
    n = x.shape[0]
    f32 = jnp.float32

    loop = jnp.arange(n, dtype=jnp.int32)
    src = jnp.concatenate([edge_index[0], loop])
    dst = jnp.concatenate([edge_index[1], loop])
    dsort, ssort = lax.sort((dst, src), num_keys=1)
    e2 = src.shape[0]
    padi = (jnp.arange(4 * KE, dtype=jnp.int32) * 997) % NN
    src_p = jnp.concatenate([ssort, padi])
    dst_p = jnp.concatenate([dsort, jnp.full((4 * KE,), BIGDST, jnp.int32)])
    marks = jnp.arange(NCHUNK + 1, dtype=jnp.int32) * CHUNK
    bnd = jnp.searchsorted(dsort, marks, side="left").astype(jnp.int32)
    bnd_p = jnp.concatenate(
        [bnd, jnp.full((BNDP - NCHUNK - 1,), e2, jnp.int32)])

    i64 = jnp.eye(64, dtype=f32)
    p1 = jnp.concatenate([i64, _sel_cat(asrc1, 64), _sel_cat(adst1, 64),
                          jnp.zeros((64, WROW - 72), f32)], axis=1)
    p2 = jnp.concatenate([i64, _sel_cat(asrc2, 64), _sel_cat(adst2, 64),
                          jnp.zeros((64, WROW - 72), f32)], axis=1)
    i42 = jnp.eye(42, dtype=f32)
    p3 = jnp.concatenate([i42, _sel_cat(asrc3, 42), _sel_cat(adst3, 42),
                          jnp.zeros((42, WROW - 54), f32)], axis=1)
    rep16 = jnp.kron(jnp.eye(4, dtype=f32), jnp.ones((1, 16), f32))
    rext = jnp.concatenate([jnp.zeros((64, 64), f32), rep16,
                            jnp.zeros((WROW - 68, 64), f32)], axis=0)
    rep7 = jnp.kron(jnp.eye(6, dtype=f32), jnp.ones((1, 7), f32))
    r3 = jnp.concatenate([jnp.zeros((48, 42), f32), rep7,
                          jnp.zeros((WROW - 54, 42), f32)], axis=0)
    m6 = jnp.kron(jnp.ones((6, 1), f32) / 6.0, jnp.eye(7, dtype=f32))
    zrs = jnp.zeros((ACC_R, WROW), f32)


    grid = (NN // MB,)
    row_spec = pl.BlockSpec((MB, WROW), lambda i: (i, 0))

    hext1 = pl.pallas_call(
        _tc1_body, grid=grid,
        in_specs=[pl.BlockSpec((MB, FIN), lambda i: (i, 0)),
                  _full_spec(FIN, 64), _full_spec(64, WROW)],
        out_specs=row_spec,
        out_shape=jax.ShapeDtypeStruct((NN, WROW), f32),
    )(x, W1, p1)

    agg1 = _sc_edge_64(hext1, src_p, dst_p, bnd_p, zrs)

    hext2 = pl.pallas_call(
        _tcmid_body, grid=grid,
        in_specs=[row_spec, _full_spec(WROW, 64), _full_spec(1, 64),
                  _full_spec(64, 64), _full_spec(64, WROW)],
        out_specs=row_spec,
        out_shape=jax.ShapeDtypeStruct((NN, WROW), f32),
    )(agg1, rext, b1.reshape(1, 64), W2, p2)

    agg2 = _sc_edge_64(hext2, src_p, dst_p, bnd_p, zrs)

    hext3 = pl.pallas_call(
        _tcmid_body, grid=grid,
        in_specs=[row_spec, _full_spec(WROW, 64), _full_spec(1, 64),
                  _full_spec(64, 42), _full_spec(42, WROW)],
        out_specs=row_spec,
        out_shape=jax.ShapeDtypeStruct((NN, WROW), f32),
    )(agg2, rext, b2.reshape(1, 64), W3, p3)

    agg3 = _sc_edge_42(hext3, src_p, dst_p, bnd_p, zrs)

    out = pl.pallas_call(
        _tcfin_body, grid=grid,
        in_specs=[row_spec, _full_spec(WROW, 42), _full_spec(42, 7),
                  _full_spec(1, 7)],
        out_specs=pl.BlockSpec((MB, 7), lambda i: (i, 0)),
        out_shape=jax.ShapeDtypeStruct((NN, 7), f32),
    )(agg3, r3, m6, b3.reshape(1, 7))

    return out
